# baseline (speedup 1.0000x reference)
import jax
import jax.numpy as jnp
from jax.experimental import pallas as pl
from jax.experimental.pallas import tpu as pltpu

_CHUNKS = (256, 512, 1024, 2560, 4096, 4096, 2048, 1024, 512, 256)


def _copy_manual(z_ref, o_ref, buf, rsem, wsem):
    n = len(_CHUNKS)
    bases = [sum(_CHUNKS[:i]) for i in range(n)]
    reads = [
        pltpu.make_async_copy(
            z_ref.at[pl.ds(bases[i], _CHUNKS[i]), :],
            buf.at[pl.ds(bases[i], _CHUNKS[i]), :],
            rsem.at[i],
        )
        for i in range(n)
    ]
    writes = [
        pltpu.make_async_copy(
            buf.at[pl.ds(bases[i], _CHUNKS[i]), :],
            o_ref.at[pl.ds(bases[i], _CHUNKS[i]), :],
            wsem.at[i],
        )
        for i in range(n)
    ]
    for r in reads:
        r.start()
    for i in range(n):
        reads[i].wait()
        writes[i].start()
    for w in writes:
        w.wait()


def kernel(z_base_global, domain_ids):
    del domain_ids
    rows, cols = z_base_global.shape
    return pl.pallas_call(
        _copy_manual,
        in_specs=[pl.BlockSpec(memory_space=pl.ANY)],
        out_specs=pl.BlockSpec(memory_space=pl.ANY),
        out_shape=jax.ShapeDtypeStruct((rows, cols), z_base_global.dtype),
        scratch_shapes=[
            pltpu.VMEM((rows, cols), z_base_global.dtype),
            pltpu.SemaphoreType.DMA((len(_CHUNKS),)),
            pltpu.SemaphoreType.DMA((len(_CHUNKS),)),
        ],
    )(z_base_global)


# speedup vs baseline: 1.0045x; 1.0045x over previous
import jax
import jax.numpy as jnp
from jax.experimental import pallas as pl
from jax.experimental.pallas import tpu as pltpu

_RAMP = (2, 4, 8, 16, 16, 8, 4, 4, 2)


def _chunk_rows(rows):
    if rows % 64 == 0:
        return tuple(rows // 64 * r for r in _RAMP)
    return (rows,)


def _make_body(chunks):
    n = len(chunks)
    bases = [sum(chunks[:i]) for i in range(n)]

    def _copy_manual(z_ref, o_ref, buf, rsem, wsem):
        reads = [
            pltpu.make_async_copy(
                z_ref.at[pl.ds(bases[i], chunks[i]), :],
                buf.at[pl.ds(bases[i], chunks[i]), :],
                rsem.at[i],
            )
            for i in range(n)
        ]
        writes = [
            pltpu.make_async_copy(
                buf.at[pl.ds(bases[i], chunks[i]), :],
                o_ref.at[pl.ds(bases[i], chunks[i]), :],
                wsem.at[i],
            )
            for i in range(n)
        ]
        for r in reads:
            r.start()
        for i in range(n):
            reads[i].wait()
            writes[i].start()
        for w in writes:
            w.wait()

    return _copy_manual


def kernel(z_base_global, domain_ids):
    del domain_ids
    rows, cols = z_base_global.shape
    chunks = _chunk_rows(rows)
    return pl.pallas_call(
        _make_body(chunks),
        in_specs=[pl.BlockSpec(memory_space=pl.ANY)],
        out_specs=pl.BlockSpec(memory_space=pl.ANY),
        out_shape=jax.ShapeDtypeStruct((rows, cols), z_base_global.dtype),
        scratch_shapes=[
            pltpu.VMEM((rows, cols), z_base_global.dtype),
            pltpu.SemaphoreType.DMA((len(chunks),)),
            pltpu.SemaphoreType.DMA((len(chunks),)),
        ],
    )(z_base_global)
